# baseline (speedup 1.0000x reference)
import functools

import jax
import jax.numpy as jnp
from jax import lax
from jax.experimental import pallas as pl
from jax.experimental.pallas import tpu as pltpu
from jax.experimental.pallas import tpu_sc as plsc

N = 10000
E = 320000
H = 128

NC = 2
NS = 16
NW = NC * NS
E_PER_W = E // NW
CHUNK = 80
NBUF = 3
SB = 6
E_W_A = 10080
E_PAD_A = NW * E_W_A
NBLKS_A = E_W_A // (SB * CHUNK)
N_PAD = 10240
ROWS_PER_TILE = N_PAD // NS
DEG_W = 128
CHUNK_D = 128
E_W_PAD = 10112
E_PAD = NW * E_W_PAD
N_CHUNKS_D = E_W_PAD // CHUNK_D
DEG_LAG = 16


def _sc_agg_body(h_hbm, ei_hbm, zrows_hbm, aggp_hbm, srcb, dstb, acc_sh,
                 *ring):
    rows = ring[:NBUF]
    gsems = ring[NBUF:2 * NBUF]
    ssems = ring[2 * NBUF:3 * NBUF]
    isem = ring[3 * NBUF]
    c = lax.axis_index("c")
    s = lax.axis_index("s")
    w = c * NS + s

    pltpu.sync_copy(zrows_hbm, acc_sh.at[pl.ds(s * ROWS_PER_TILE, ROWS_PER_TILE)])
    plsc.subcore_barrier()

    def ga(r, p, b):
        pltpu.async_copy(h_hbm.at[srcb.at[r, p]], rows[b], gsems[b])

    def gw(b):
        pltpu.make_async_copy(h_hbm.at[srcb.at[0, 0]], rows[b],
                              gsems[b]).wait()

    def sc(r, p, b):
        pltpu.async_copy(rows[b], acc_sh.at[dstb.at[r, p]], ssems[b],
                         add=True)

    def sw(b):
        pltpu.make_async_copy(rows[b], acc_sh.at[dstb.at[0, 0]],
                              ssems[b]).wait()

    def ipre(m, r):
        pltpu.async_copy(ei_hbm.at[0, w, m], srcb.at[r], isem)
        pltpu.async_copy(ei_hbm.at[1, w, m], dstb.at[r], isem)

    def iw():
        pltpu.make_async_copy(ei_hbm.at[0, w, 0], srcb.at[0], isem).wait()
        pltpu.make_async_copy(ei_hbm.at[1, w, 0], dstb.at[0], isem).wait()

    pltpu.sync_copy(ei_hbm.at[0, w, 0], srcb.at[0])
    pltpu.sync_copy(ei_hbm.at[1, w, 0], dstb.at[0])
    ipre(1, 1)
    ga(0, 0, 0)
    ga(0, 1, 1)
    ga(0, 2, 2)
    gw(0); sc(0, 0, 0)
    gw(1); sc(0, 1, 1); sw(0); ga(0, 3, 0)
    gw(2); sc(0, 2, 2); sw(1); ga(0, 4, 1)
    gw(0); sc(0, 3, 0); sw(2); ga(0, 5, 2)
    iw()
    gw(1); sc(0, 4, 1); sw(0); ga(1, 0, 0)
    gw(2); sc(0, 5, 2); sw(1); ga(1, 1, 1)

    @pl.loop(1, NBLKS_A - 1)
    def _(m):
        r = lax.rem(m, 2)
        rn = 1 - r
        gw(0); sc(r, 0, 0); sw(2); ga(r, 2, 2)
        ipre(m + 1, rn)
        gw(1); sc(r, 1, 1); sw(0); ga(r, 3, 0)
        gw(2); sc(r, 2, 2); sw(1); ga(r, 4, 1)
        gw(0); sc(r, 3, 0); sw(2); ga(r, 5, 2)
        iw()
        gw(1); sc(r, 4, 1); sw(0); ga(rn, 0, 0)
        gw(2); sc(r, 5, 2); sw(1); ga(rn, 1, 1)

    gw(0); sc(0, 0, 0); sw(2); ga(0, 2, 2)
    gw(1); sc(0, 1, 1); sw(0); ga(0, 3, 0)
    gw(2); sc(0, 2, 2); sw(1); ga(0, 4, 1)
    gw(0); sc(0, 3, 0); sw(2); ga(0, 5, 2)
    gw(1); sc(0, 4, 1); sw(0)
    gw(2); sc(0, 5, 2); sw(1)
    sw(2)

    plsc.subcore_barrier()

    sl = pl.ds(s * ROWS_PER_TILE, ROWS_PER_TILE)
    pltpu.sync_copy(acc_sh.at[sl], aggp_hbm.at[c, sl])


@functools.lru_cache(maxsize=None)
def _get_sc_agg():
    return pl.kernel(
        _sc_agg_body,
        out_type=jax.ShapeDtypeStruct((NC, N_PAD, H), jnp.float32),
        mesh=plsc.VectorSubcoreMesh(core_axis_name="c", subcore_axis_name="s"),
        scratch_types=[
            pltpu.VMEM((2, SB, CHUNK), jnp.int32),
            pltpu.VMEM((2, SB, CHUNK), jnp.int32),
            pltpu.VMEM_SHARED((N_PAD, H), jnp.float32),
        ] + [pltpu.VMEM((CHUNK, H), jnp.float32)] * NBUF
          + [pltpu.SemaphoreType.DMA] * (3 * NBUF + 1),
        name="sc_agg",
    )


def _sc_deg_body(ei_hbm, zdeg_hbm, ones_hbm, degp_hbm, dst_v, ones_v, deg_sh,
                 sem):
    c = lax.axis_index("c")
    s = lax.axis_index("s")
    w = c * NS + s

    pltpu.sync_copy(zdeg_hbm, deg_sh.at[pl.ds(s * ROWS_PER_TILE, ROWS_PER_TILE)])
    pltpu.sync_copy(ei_hbm.at[1, w], dst_v)
    pltpu.sync_copy(ones_hbm, ones_v)
    plsc.subcore_barrier()

    @pl.loop(0, N_CHUNKS_D)
    def _(j):
        pltpu.async_copy(ones_v, deg_sh.at[dst_v.at[j]], sem, add=True)

        @pl.when(j >= DEG_LAG)
        def _():
            pltpu.make_async_copy(ones_v, deg_sh.at[dst_v.at[0]], sem).wait()

    @pl.loop(0, DEG_LAG)
    def _(j):
        pltpu.make_async_copy(ones_v, deg_sh.at[dst_v.at[0]], sem).wait()

    plsc.subcore_barrier()
    sl = pl.ds(s * ROWS_PER_TILE, ROWS_PER_TILE)
    pltpu.sync_copy(deg_sh.at[sl], degp_hbm.at[c, sl])


@functools.lru_cache(maxsize=None)
def _get_sc_deg():
    return pl.kernel(
        _sc_deg_body,
        out_type=jax.ShapeDtypeStruct((NC, N_PAD, DEG_W), jnp.float32),
        mesh=plsc.VectorSubcoreMesh(core_axis_name="c", subcore_axis_name="s"),
        scratch_types=[
            pltpu.VMEM((N_CHUNKS_D, CHUNK_D), jnp.int32),
            pltpu.VMEM((CHUNK_D, DEG_W), jnp.float32),
            pltpu.VMEM_SHARED((N_PAD, DEG_W), jnp.float32),
            pltpu.SemaphoreType.DMA,
        ],
        name="sc_deg",
    )


def _tc_layer_body(final, h_ref, a_ref, d_ref, ws_ref, wn_ref, b_ref,
                   g_ref, be_ref, wc_ref, bc_ref, o_ref):
    agg = a_ref[0, 0:N, :] + a_ref[1, 0:N, :]
    deg = d_ref[0, 0:N, 0:1] + d_ref[1, 0:N, 0:1]
    mean = agg / jnp.maximum(deg, 1.0)
    h = h_ref[...]
    z = (jnp.dot(h, ws_ref[...], preferred_element_type=jnp.float32)
         + jnp.dot(mean, wn_ref[...], preferred_element_type=jnp.float32)
         + b_ref[...])
    m = jnp.mean(z, axis=0, keepdims=True)
    v = jnp.mean((z - m) * (z - m), axis=0, keepdims=True)
    zn = (z - m) * lax.rsqrt(v + 1e-5)
    act = jnp.maximum(g_ref[...] * zn + be_ref[...], 0.0)
    if final:
        o_ref[...] = (jnp.dot(act, wc_ref[...], preferred_element_type=jnp.float32)
                      + bc_ref[...])
    else:
        o_ref[...] = act


def _tc_layer(h, aggp, degp, Ws, Wn, b, g, be, wc_pad, bc_pad, final):
    return pl.pallas_call(
        functools.partial(_tc_layer_body, final),
        out_shape=jax.ShapeDtypeStruct((N, H), jnp.float32),
    )(h, aggp, degp, Ws, Wn, b.reshape(1, H), g.reshape(1, H),
      be.reshape(1, H), wc_pad, bc_pad)


def kernel(features, edge_index, Ws0, Wn0, b0, g0, be0, Ws1, Wn1, b1, g1,
           be1, Ws2, Wn2, b2, g2, be2, Wc, bc):
    pad_a = jnp.concatenate(
        [jnp.zeros((1, E_PAD_A - E), jnp.int32),
         jnp.full((1, E_PAD_A - E), N, jnp.int32)], axis=0)
    ei = jnp.concatenate([edge_index, pad_a], axis=1).reshape(
        2, NW, NBLKS_A, SB, CHUNK)
    pad = jnp.concatenate(
        [jnp.zeros((1, E_PAD - E), jnp.int32),
         jnp.full((1, E_PAD - E), N, jnp.int32)], axis=0)
    ei_deg = jnp.concatenate([edge_index, pad], axis=1)
    ei_flat = ei_deg.reshape(2, NW, N_CHUNKS_D, CHUNK_D)
    zrows = jnp.zeros((ROWS_PER_TILE, H), jnp.float32)
    ones = jnp.ones((CHUNK_D, DEG_W), jnp.float32)
    wc_pad = jnp.zeros((H, H), jnp.float32).at[:, :Wc.shape[1]].set(Wc)
    bc_pad = jnp.zeros((1, H), jnp.float32).at[0, :bc.shape[0]].set(bc)

    h = features
    degp = _get_sc_deg()(ei_flat, zrows, ones)
    aggp = _get_sc_agg()(h, ei, zrows)
    h = _tc_layer(h, aggp, degp, Ws0, Wn0, b0, g0, be0, wc_pad, bc_pad, False)
    aggp = _get_sc_agg()(h, ei, zrows)
    h = _tc_layer(h, aggp, degp, Ws1, Wn1, b1, g1, be1, wc_pad, bc_pad, False)
    aggp = _get_sc_agg()(h, ei, zrows)
    out = _tc_layer(h, aggp, degp, Ws2, Wn2, b2, g2, be2, wc_pad, bc_pad, True)
    return out[:, :Wc.shape[1]]


# speedup vs baseline: 1.8033x; 1.8033x over previous
import functools

import jax
import jax.numpy as jnp
from jax import lax
from jax.experimental import pallas as pl
from jax.experimental.pallas import tpu as pltpu
from jax.experimental.pallas import tpu_sc as plsc

N = 10000
E = 320000
H = 128

NC = 2
NS = 16
NW = NC * NS
E_PER_W = E // NW
CHUNK = 40
NBUF = 5
SB = 10
E_W_A = 10000
NBLKS_A = E_W_A // (SB * CHUNK)
N_PAD = 10240
ROWS_PER_TILE = N_PAD // NS
DEG_W = 128
CHUNK_D = 128
E_W_PAD = 10112
E_PAD = NW * E_W_PAD
N_CHUNKS_D = E_W_PAD // CHUNK_D
DEG_LAG = 16


def _sc_agg_body(h_hbm, ei_hbm, zrows_hbm, aggp_hbm, srcb, dstb, acc_sh,
                 *ring):
    rows = ring[:NBUF]
    gsems = ring[NBUF:2 * NBUF]
    ssems = ring[2 * NBUF:3 * NBUF]
    isem = ring[3 * NBUF]
    c = lax.axis_index("c")
    s = lax.axis_index("s")
    w = c * NS + s

    pltpu.sync_copy(zrows_hbm, acc_sh.at[pl.ds(s * ROWS_PER_TILE, ROWS_PER_TILE)])
    plsc.subcore_barrier()

    def ga(r, p, b):
        pltpu.async_copy(h_hbm.at[srcb.at[r, p]], rows[b], gsems[b])

    def gw(b):
        pltpu.make_async_copy(h_hbm.at[srcb.at[0, 0]], rows[b],
                              gsems[b]).wait()

    def sc(r, p, b):
        pltpu.async_copy(rows[b], acc_sh.at[dstb.at[r, p]], ssems[b],
                         add=True)

    def sw(b):
        pltpu.make_async_copy(rows[b], acc_sh.at[dstb.at[0, 0]],
                              ssems[b]).wait()

    def ipre(m, r):
        pltpu.async_copy(ei_hbm.at[0, w, m], srcb.at[r], isem)
        pltpu.async_copy(ei_hbm.at[1, w, m], dstb.at[r], isem)

    def iw():
        pltpu.make_async_copy(ei_hbm.at[0, w, 0], srcb.at[0], isem).wait()
        pltpu.make_async_copy(ei_hbm.at[1, w, 0], dstb.at[0], isem).wait()

    pltpu.sync_copy(ei_hbm.at[0, w, 0], srcb.at[0])
    pltpu.sync_copy(ei_hbm.at[1, w, 0], dstb.at[0])
    ipre(1, 1)
    ga(0, 0, 0)
    ga(0, 1, 1)
    ga(0, 2, 2)
    ga(0, 3, 3)
    gw(0); sc(0, 0, 0); ga(0, 4, 4)
    gw(1); sc(0, 1, 1); sw(0); ga(0, 5, 0)
    gw(2); sc(0, 2, 2); sw(1); ga(0, 6, 1)
    gw(3); sc(0, 3, 3); sw(2); ga(0, 7, 2)
    gw(4); sc(0, 4, 4); sw(3); ga(0, 8, 3)
    gw(0); sc(0, 5, 0); sw(4); ga(0, 9, 4)
    iw()
    gw(1); sc(0, 6, 1); sw(0); ga(1, 0, 0)
    gw(2); sc(0, 7, 2); sw(1); ga(1, 1, 1)
    gw(3); sc(0, 8, 3); sw(2); ga(1, 2, 2)
    gw(4); sc(0, 9, 4); sw(3); ga(1, 3, 3)

    @pl.loop(1, NBLKS_A - 1)
    def _(m):
        r = lax.rem(m, 2)
        rn = 1 - r
        gw(0); sc(r, 0, 0); sw(4); ga(r, 4, 4)
        ipre(m + 1, rn)
        gw(1); sc(r, 1, 1); sw(0); ga(r, 5, 0)
        gw(2); sc(r, 2, 2); sw(1); ga(r, 6, 1)
        gw(3); sc(r, 3, 3); sw(2); ga(r, 7, 2)
        gw(4); sc(r, 4, 4); sw(3); ga(r, 8, 3)
        gw(0); sc(r, 5, 0); sw(4); ga(r, 9, 4)
        iw()
        gw(1); sc(r, 6, 1); sw(0); ga(rn, 0, 0)
        gw(2); sc(r, 7, 2); sw(1); ga(rn, 1, 1)
        gw(3); sc(r, 8, 3); sw(2); ga(rn, 2, 2)
        gw(4); sc(r, 9, 4); sw(3); ga(rn, 3, 3)

    gw(0); sc(0, 0, 0); sw(4); ga(0, 4, 4)
    gw(1); sc(0, 1, 1); sw(0); ga(0, 5, 0)
    gw(2); sc(0, 2, 2); sw(1); ga(0, 6, 1)
    gw(3); sc(0, 3, 3); sw(2); ga(0, 7, 2)
    gw(4); sc(0, 4, 4); sw(3); ga(0, 8, 3)
    gw(0); sc(0, 5, 0); sw(4); ga(0, 9, 4)
    gw(1); sc(0, 6, 1); sw(0)
    gw(2); sc(0, 7, 2); sw(1)
    gw(3); sc(0, 8, 3); sw(2)
    gw(4); sc(0, 9, 4); sw(3)
    sw(4)

    plsc.subcore_barrier()

    sl = pl.ds(s * ROWS_PER_TILE, ROWS_PER_TILE)
    pltpu.sync_copy(acc_sh.at[sl], aggp_hbm.at[c, sl])


@functools.lru_cache(maxsize=None)
def _get_sc_agg():
    return pl.kernel(
        _sc_agg_body,
        out_type=jax.ShapeDtypeStruct((NC, N_PAD, H), jnp.float32),
        mesh=plsc.VectorSubcoreMesh(core_axis_name="c", subcore_axis_name="s"),
        scratch_types=[
            pltpu.VMEM((2, SB, CHUNK), jnp.int32),
            pltpu.VMEM((2, SB, CHUNK), jnp.int32),
            pltpu.VMEM_SHARED((N_PAD, H), jnp.float32),
        ] + [pltpu.VMEM((CHUNK, H), jnp.float32)] * NBUF
          + [pltpu.SemaphoreType.DMA] * (3 * NBUF + 1),
        name="sc_agg",
    )


def _sc_deg_body(ei_hbm, zdeg_hbm, ones_hbm, degp_hbm, dst_v, ones_v, deg_sh,
                 sem):
    c = lax.axis_index("c")
    s = lax.axis_index("s")
    w = c * NS + s

    pltpu.sync_copy(zdeg_hbm, deg_sh.at[pl.ds(s * ROWS_PER_TILE, ROWS_PER_TILE)])
    pltpu.sync_copy(ei_hbm.at[1, w], dst_v)
    pltpu.sync_copy(ones_hbm, ones_v)
    plsc.subcore_barrier()

    @pl.loop(0, N_CHUNKS_D)
    def _(j):
        pltpu.async_copy(ones_v, deg_sh.at[dst_v.at[j]], sem, add=True)

        @pl.when(j >= DEG_LAG)
        def _():
            pltpu.make_async_copy(ones_v, deg_sh.at[dst_v.at[0]], sem).wait()

    @pl.loop(0, DEG_LAG)
    def _(j):
        pltpu.make_async_copy(ones_v, deg_sh.at[dst_v.at[0]], sem).wait()

    plsc.subcore_barrier()
    sl = pl.ds(s * ROWS_PER_TILE, ROWS_PER_TILE)
    pltpu.sync_copy(deg_sh.at[sl], degp_hbm.at[c, sl])


@functools.lru_cache(maxsize=None)
def _get_sc_deg():
    return pl.kernel(
        _sc_deg_body,
        out_type=jax.ShapeDtypeStruct((NC, N_PAD, DEG_W), jnp.float32),
        mesh=plsc.VectorSubcoreMesh(core_axis_name="c", subcore_axis_name="s"),
        scratch_types=[
            pltpu.VMEM((N_CHUNKS_D, CHUNK_D), jnp.int32),
            pltpu.VMEM((CHUNK_D, DEG_W), jnp.float32),
            pltpu.VMEM_SHARED((N_PAD, DEG_W), jnp.float32),
            pltpu.SemaphoreType.DMA,
        ],
        name="sc_deg",
    )


def _tc_layer_body(final, h_ref, a_ref, d_ref, ws_ref, wn_ref, b_ref,
                   g_ref, be_ref, wc_ref, bc_ref, o_ref):
    agg = a_ref[0, 0:N, :] + a_ref[1, 0:N, :]
    deg = d_ref[0, 0:N, 0:1] + d_ref[1, 0:N, 0:1]
    mean = agg / jnp.maximum(deg, 1.0)
    h = h_ref[...]
    z = (jnp.dot(h, ws_ref[...], preferred_element_type=jnp.float32)
         + jnp.dot(mean, wn_ref[...], preferred_element_type=jnp.float32)
         + b_ref[...])
    m = jnp.mean(z, axis=0, keepdims=True)
    v = jnp.mean((z - m) * (z - m), axis=0, keepdims=True)
    zn = (z - m) * lax.rsqrt(v + 1e-5)
    act = jnp.maximum(g_ref[...] * zn + be_ref[...], 0.0)
    if final:
        o_ref[...] = (jnp.dot(act, wc_ref[...], preferred_element_type=jnp.float32)
                      + bc_ref[...])
    else:
        o_ref[...] = act


def _tc_layer(h, aggp, degp, Ws, Wn, b, g, be, wc_pad, bc_pad, final):
    return pl.pallas_call(
        functools.partial(_tc_layer_body, final),
        out_shape=jax.ShapeDtypeStruct((N, H), jnp.float32),
    )(h, aggp, degp, Ws, Wn, b.reshape(1, H), g.reshape(1, H),
      be.reshape(1, H), wc_pad, bc_pad)


def kernel(features, edge_index, Ws0, Wn0, b0, g0, be0, Ws1, Wn1, b1, g1,
           be1, Ws2, Wn2, b2, g2, be2, Wc, bc):
    ei = edge_index.reshape(2, NW, NBLKS_A, SB, CHUNK)
    pad = jnp.concatenate(
        [jnp.zeros((1, E_PAD - E), jnp.int32),
         jnp.full((1, E_PAD - E), N, jnp.int32)], axis=0)
    ei_deg = jnp.concatenate([edge_index, pad], axis=1)
    ei_flat = ei_deg.reshape(2, NW, N_CHUNKS_D, CHUNK_D)
    zrows = jnp.zeros((ROWS_PER_TILE, H), jnp.float32)
    ones = jnp.ones((CHUNK_D, DEG_W), jnp.float32)
    wc_pad = jnp.zeros((H, H), jnp.float32).at[:, :Wc.shape[1]].set(Wc)
    bc_pad = jnp.zeros((1, H), jnp.float32).at[0, :bc.shape[0]].set(bc)

    h = features
    degp = _get_sc_deg()(ei_flat, zrows, ones)
    aggp = _get_sc_agg()(h, ei, zrows)
    h = _tc_layer(h, aggp, degp, Ws0, Wn0, b0, g0, be0, wc_pad, bc_pad, False)
    aggp = _get_sc_agg()(h, ei, zrows)
    h = _tc_layer(h, aggp, degp, Ws1, Wn1, b1, g1, be1, wc_pad, bc_pad, False)
    aggp = _get_sc_agg()(h, ei, zrows)
    out = _tc_layer(h, aggp, degp, Ws2, Wn2, b2, g2, be2, wc_pad, bc_pad, True)
    return out[:, :Wc.shape[1]]
